# baseline (speedup 1.0000x reference)
import functools

import jax
import jax.numpy as jnp
from jax import lax
from jax.experimental import pallas as pl
from jax.experimental.pallas import tpu as pltpu
from jax.experimental.pallas import tpu_sc as plsc

N_NODES = 10000
N_EDGES = 320000
D = 128
NUM_GRAPHS = 16

NC, NS = 2, 16
NTILES = NC * NS
CH = 128
CPT = 80
E_PAD = NTILES * CPT * CH
DUMMY = N_NODES
NP = 10112
ZR = NP // NS

@functools.cache
def _sc_kernels():
    mesh = plsc.VectorSubcoreMesh(
        core_axis_name="c", subcore_axis_name="s", num_cores=NC, num_subcores=NS
    )

    @functools.partial(
        pl.kernel,
        out_type=jax.ShapeDtypeStruct((NC, NP, D), jnp.float32),
        mesh=mesh,
        scratch_types=(
            pltpu.VMEM((CPT // 2, CH), jnp.int32),
            pltpu.VMEM((CPT // 2, CH), jnp.int32),
            pltpu.VMEM((CH, D), jnp.float32),
            pltpu.VMEM((CH, D), jnp.float32),
            pltpu.VMEM_SHARED((NP, D), jnp.float32),
            pltpu.SemaphoreType.DMA,
            pltpu.SemaphoreType.DMA,
        ),
    )
    def _edge_scatter(y_hbm, src_hbm, dst_hbm, z_hbm, out_hbm,
                      src_v, dst_v, rows0, rows1, acc, semg0, semg1):
        cid = lax.axis_index("c")
        sid = lax.axis_index("s")
        wid = cid * NS + sid
        r0 = sid * ZR
        hcpt = CPT // 2
        pltpu.sync_copy(z_hbm, acc.at[pl.ds(r0, ZR)])
        plsc.subcore_barrier()

        def run_half(h):
            pltpu.sync_copy(src_hbm.at[wid, pl.ds(h * hcpt, hcpt)], src_v)
            pltpu.sync_copy(dst_hbm.at[wid, pl.ds(h * hcpt, hcpt)], dst_v)
            pltpu.async_copy(y_hbm.at[src_v.at[0]], rows0, semg0)

            def body(p, carry):
                j0 = 2 * p
                pltpu.async_copy(y_hbm.at[src_v.at[j0 + 1]], rows1, semg1)
                pltpu.make_async_copy(y_hbm.at[src_v.at[0]], rows0, semg0).wait()
                pltpu.sync_copy(rows0, acc.at[dst_v.at[j0]], add=True)
                jn = lax.min(j0 + 2, hcpt - 1)
                pltpu.async_copy(y_hbm.at[src_v.at[jn]], rows0, semg0)
                pltpu.make_async_copy(y_hbm.at[src_v.at[0]], rows1, semg1).wait()
                pltpu.sync_copy(rows1, acc.at[dst_v.at[j0 + 1]], add=True)
                return carry

            lax.fori_loop(0, hcpt // 2, body, 0)
            pltpu.make_async_copy(y_hbm.at[src_v.at[0]], rows0, semg0).wait()

        run_half(0)
        run_half(1)
        plsc.subcore_barrier()
        pltpu.sync_copy(acc.at[pl.ds(r0, ZR)], out_hbm.at[cid, pl.ds(r0, ZR)])

    @functools.partial(
        pl.kernel,
        out_type=jax.ShapeDtypeStruct((NC, NP, 16), jnp.float32),
        mesh=mesh,
        scratch_types=(
            pltpu.VMEM((CPT, CH), jnp.int32),
            pltpu.VMEM((CH, 16), jnp.float32),
            pltpu.VMEM_SHARED((NP, 16), jnp.float32),
        ),
    )
    def _deg_scatter(dst_hbm, z_hbm, ones_hbm, out_hbm, dsts, ones, acc):
        cid = lax.axis_index("c")
        sid = lax.axis_index("s")
        wid = cid * NS + sid
        r0 = sid * ZR
        pltpu.sync_copy(z_hbm, acc.at[pl.ds(r0, ZR)])
        pltpu.sync_copy(dst_hbm.at[wid], dsts)
        pltpu.sync_copy(ones_hbm, ones)
        plsc.subcore_barrier()

        def body(j, carry):
            pltpu.sync_copy(ones, acc.at[dsts.at[j]], add=True)
            return carry

        lax.fori_loop(0, CPT, body, 0)
        plsc.subcore_barrier()
        pltpu.sync_copy(acc.at[pl.ds(r0, ZR)], out_hbm.at[cid, pl.ds(r0, ZR)])

    return _edge_scatter, _deg_scatter


def _dis_of(degp):
    deg = degp[0, :, 0:1] + degp[1, :, 0:1] + 1.0
    return 1.0 / jnp.sqrt(deg)


def _enc_body(x_ref, w_ref, b_ref, o_ref):
    v = jnp.dot(x_ref[...], w_ref[...], preferred_element_type=jnp.float32)
    v = v + b_ref[...]
    o_ref[...] = jnp.where(v > 0, v, jnp.exp(jnp.minimum(v, 0.0)) - 1.0)


def _pre_body(h_ref, w_ref, degp_ref, o_ref):
    dis = _dis_of(degp_ref[...])
    o_ref[...] = (
        jnp.dot(h_ref[...], w_ref[...], preferred_element_type=jnp.float32) * dis
    )


def _postpre_body(p_ref, y_ref, degp_ref, b_ref, g_ref, be_ref, w_ref, o_ref):
    dis = _dis_of(degp_ref[...])
    accv = p_ref[0] + p_ref[1] + y_ref[...]
    t = jnp.maximum(accv * dis + b_ref[...], 0.0)
    mu = jnp.mean(t, axis=-1, keepdims=True)
    var = jnp.mean((t - mu) ** 2, axis=-1, keepdims=True)
    h = (t - mu) / jnp.sqrt(var + 1e-5) * g_ref[...] + be_ref[...]
    o_ref[...] = (
        jnp.dot(h, w_ref[...], preferred_element_type=jnp.float32) * dis
    )


def _final_body(p_ref, y_ref, degp_ref, b_ref, g_ref, be_ref, bat_ref,
                pw_ref, pb_ref, o_ref, sums, cnts):
    i = pl.program_id(0)

    @pl.when(i == 0)
    def _():
        sums[...] = jnp.zeros_like(sums)
        cnts[...] = jnp.zeros_like(cnts)

    dis = _dis_of(degp_ref[...])
    accv = p_ref[0] + p_ref[1] + y_ref[...]
    t = jnp.maximum(accv * dis + b_ref[...], 0.0)
    mu = jnp.mean(t, axis=-1, keepdims=True)
    var = jnp.mean((t - mu) ** 2, axis=-1, keepdims=True)
    h = (t - mu) / jnp.sqrt(var + 1e-5) * g_ref[...] + be_ref[...]

    bb = bat_ref[0]
    gid = lax.broadcasted_iota(jnp.int32, (NUM_GRAPHS, bb.shape[1]), 0)
    oh = (gid == bb).astype(jnp.float32)
    sums[...] += jnp.dot(oh, h, preferred_element_type=jnp.float32)
    cnts[...] += jnp.broadcast_to(
        jnp.sum(oh, axis=1, keepdims=True), (NUM_GRAPHS, D)
    )

    @pl.when(i == pl.num_programs(0) - 1)
    def _():
        pooled = sums[...] / jnp.maximum(cnts[...], 1.0)
        logits = (
            jnp.dot(pooled, pw_ref[...], preferred_element_type=jnp.float32)
            + pb_ref[...]
        )
        o_ref[...] = jax.nn.sigmoid(logits)


_R = 2528

_enc = pl.pallas_call(
    _enc_body,
    grid=(10,),
    in_specs=[
        pl.BlockSpec((1000, D), lambda i: (i, 0)),
        pl.BlockSpec((D, D), lambda i: (0, 0)),
        pl.BlockSpec((1, D), lambda i: (0, 0)),
    ],
    out_specs=pl.BlockSpec((1000, D), lambda i: (i, 0)),
    out_shape=jax.ShapeDtypeStruct((N_NODES, D), jnp.float32),
)

_pre = pl.pallas_call(
    _pre_body,
    grid=(NP // _R,),
    in_specs=[
        pl.BlockSpec((_R, D), lambda i: (i, 0)),
        pl.BlockSpec((D, D), lambda i: (0, 0)),
        pl.BlockSpec((NC, _R, 16), lambda i: (0, i, 0)),
    ],
    out_specs=pl.BlockSpec((_R, D), lambda i: (i, 0)),
    out_shape=jax.ShapeDtypeStruct((NP, D), jnp.float32),
)

_postpre = pl.pallas_call(
    _postpre_body,
    grid=(NP // _R,),
    in_specs=[
        pl.BlockSpec((NC, _R, D), lambda i: (0, i, 0)),
        pl.BlockSpec((_R, D), lambda i: (i, 0)),
        pl.BlockSpec((NC, _R, 16), lambda i: (0, i, 0)),
        pl.BlockSpec((1, D), lambda i: (0, 0)),
        pl.BlockSpec((1, D), lambda i: (0, 0)),
        pl.BlockSpec((1, D), lambda i: (0, 0)),
        pl.BlockSpec((D, D), lambda i: (0, 0)),
    ],
    out_specs=pl.BlockSpec((_R, D), lambda i: (i, 0)),
    out_shape=jax.ShapeDtypeStruct((NP, D), jnp.float32),
)

_RB = 1000

_final = pl.pallas_call(
    _final_body,
    grid=(N_NODES // _RB,),
    in_specs=[
        pl.BlockSpec((NC, _RB, D), lambda i: (0, i, 0)),
        pl.BlockSpec((_RB, D), lambda i: (i, 0)),
        pl.BlockSpec((NC, _RB, 16), lambda i: (0, i, 0)),
        pl.BlockSpec((1, D), lambda i: (0, 0)),
        pl.BlockSpec((1, D), lambda i: (0, 0)),
        pl.BlockSpec((1, D), lambda i: (0, 0)),
        pl.BlockSpec((1, 1, _RB), lambda i: (i, 0, 0)),
        pl.BlockSpec((D, D), lambda i: (0, 0)),
        pl.BlockSpec((1, D), lambda i: (0, 0)),
    ],
    out_specs=pl.BlockSpec((NUM_GRAPHS, D), lambda i: (0, 0)),
    out_shape=jax.ShapeDtypeStruct((NUM_GRAPHS, D), jnp.float32),
    scratch_shapes=[
        pltpu.VMEM((NUM_GRAPHS, D), jnp.float32),
        pltpu.VMEM((NUM_GRAPHS, D), jnp.float32),
    ],
)


def kernel(x, edge_index, batch, enc_W, enc_b, W1, b1, g1, be1,
           W2, b2, g2, be2, W3, b3, g3, be3, pred_W, pred_b):
    src = edge_index[0].astype(jnp.int32)
    dst = edge_index[1].astype(jnp.int32)
    pad = jnp.full((E_PAD - N_EDGES,), DUMMY, jnp.int32)
    src_p = jnp.concatenate([src, pad]).reshape(NTILES, CPT, CH)
    dst_p = jnp.concatenate([dst, pad]).reshape(NTILES, CPT, CH)
    zrows = jnp.zeros((ZR, D), jnp.float32)
    z16 = jnp.zeros((ZR, 16), jnp.float32)
    ones16 = jnp.ones((CH, 16), jnp.float32)

    _edge_scatter, _deg_scatter = _sc_kernels()
    degp = _deg_scatter(dst_p, z16, ones16)
    h = _enc(x, enc_W, enc_b.reshape(1, D))
    hp = jnp.pad(h, ((0, NP - N_NODES), (0, 0)))

    y = _pre(hp, W1, degp)
    P = _edge_scatter(y, src_p, dst_p, zrows)
    y = _postpre(P, y, degp, b1.reshape(1, D), g1.reshape(1, D),
                 be1.reshape(1, D), W2)
    P = _edge_scatter(y, src_p, dst_p, zrows)
    y = _postpre(P, y, degp, b2.reshape(1, D), g2.reshape(1, D),
                 be2.reshape(1, D), W3)
    P = _edge_scatter(y, src_p, dst_p, zrows)

    pWp = jnp.zeros((D, D), jnp.float32).at[:, :2].set(pred_W)
    pbp = jnp.zeros((1, D), jnp.float32).at[0, :2].set(pred_b)
    batp = batch.astype(jnp.int32).reshape(N_NODES // _RB, 1, _RB)
    probs = _final(P[:, :N_NODES], y[:N_NODES], degp[:, :N_NODES],
                   b3.reshape(1, D), g3.reshape(1, D), be3.reshape(1, D),
                   batp, pWp, pbp)
    return probs[:, :2]


# speedup vs baseline: 1.4192x; 1.4192x over previous
import functools

import jax
import jax.numpy as jnp
from jax import lax
from jax.experimental import pallas as pl
from jax.experimental.pallas import tpu as pltpu
from jax.experimental.pallas import tpu_sc as plsc

N_NODES = 10000
N_EDGES = 320000
D = 128
NUM_GRAPHS = 16

NC, NS = 2, 16
NTILES = NC * NS
CH = 128
CPT = 79
E_PAD = NTILES * CPT * CH
DUMMY = N_NODES
NP = 10112
ZR = NP // NS

@functools.cache
def _sc_kernels():
    mesh = plsc.VectorSubcoreMesh(
        core_axis_name="c", subcore_axis_name="s", num_cores=NC, num_subcores=NS
    )

    @functools.partial(
        pl.kernel,
        out_type=jax.ShapeDtypeStruct((NC, NP, D), jnp.float32),
        mesh=mesh,
        scratch_types=(
            pltpu.VMEM((CPT, CH), jnp.int32),
            pltpu.VMEM((CPT, CH), jnp.int32),
            pltpu.VMEM((CH, D), jnp.float32),
            pltpu.VMEM_SHARED((NP, D), jnp.float32),
            pltpu.SemaphoreType.DMA,
        ),
    )
    def _edge_scatter(y_hbm, src_hbm, dst_hbm, z_hbm, out_hbm,
                      srcs, dsts, rows, acc, sem):
        cid = lax.axis_index("c")
        sid = lax.axis_index("s")
        wid = cid * NS + sid
        r0 = sid * ZR
        pltpu.sync_copy(z_hbm, acc.at[pl.ds(r0, ZR)])
        pltpu.sync_copy(src_hbm.at[wid], srcs)
        pltpu.sync_copy(dst_hbm.at[wid], dsts)
        plsc.subcore_barrier()

        def body(j, carry):
            pltpu.async_copy(y_hbm.at[srcs.at[j]], rows, sem).wait()
            pltpu.sync_copy(rows, acc.at[dsts.at[j]], add=True)
            return carry

        lax.fori_loop(0, CPT, body, 0)
        plsc.subcore_barrier()
        pltpu.sync_copy(acc.at[pl.ds(r0, ZR)], out_hbm.at[cid, pl.ds(r0, ZR)])

    @functools.partial(
        pl.kernel,
        out_type=jax.ShapeDtypeStruct((NC, NP, 16), jnp.float32),
        mesh=mesh,
        scratch_types=(
            pltpu.VMEM((CPT, CH), jnp.int32),
            pltpu.VMEM((CH, 16), jnp.float32),
            pltpu.VMEM_SHARED((NP, 16), jnp.float32),
        ),
    )
    def _deg_scatter(dst_hbm, z_hbm, ones_hbm, out_hbm, dsts, ones, acc):
        cid = lax.axis_index("c")
        sid = lax.axis_index("s")
        wid = cid * NS + sid
        r0 = sid * ZR
        pltpu.sync_copy(z_hbm, acc.at[pl.ds(r0, ZR)])
        pltpu.sync_copy(dst_hbm.at[wid], dsts)
        pltpu.sync_copy(ones_hbm, ones)
        plsc.subcore_barrier()

        def body(j, carry):
            pltpu.sync_copy(ones, acc.at[dsts.at[j]], add=True)
            return carry

        lax.fori_loop(0, CPT, body, 0)
        plsc.subcore_barrier()
        pltpu.sync_copy(acc.at[pl.ds(r0, ZR)], out_hbm.at[cid, pl.ds(r0, ZR)])

    return _edge_scatter, _deg_scatter


def _dis_of(degp):
    deg = degp[0, :, 0:1] + degp[1, :, 0:1] + 1.0
    return 1.0 / jnp.sqrt(deg)


def _enc_body(x_ref, w_ref, b_ref, o_ref):
    v = jnp.dot(x_ref[...], w_ref[...], preferred_element_type=jnp.float32)
    v = v + b_ref[...]
    o_ref[...] = jnp.where(v > 0, v, jnp.exp(jnp.minimum(v, 0.0)) - 1.0)


def _pre_body(h_ref, w_ref, degp_ref, o_ref):
    dis = _dis_of(degp_ref[...])
    o_ref[...] = (
        jnp.dot(h_ref[...], w_ref[...], preferred_element_type=jnp.float32) * dis
    )


def _postpre_body(p_ref, y_ref, degp_ref, b_ref, g_ref, be_ref, w_ref, o_ref):
    dis = _dis_of(degp_ref[...])
    accv = p_ref[0] + p_ref[1] + y_ref[...]
    t = jnp.maximum(accv * dis + b_ref[...], 0.0)
    mu = jnp.mean(t, axis=-1, keepdims=True)
    var = jnp.mean((t - mu) ** 2, axis=-1, keepdims=True)
    h = (t - mu) / jnp.sqrt(var + 1e-5) * g_ref[...] + be_ref[...]
    o_ref[...] = (
        jnp.dot(h, w_ref[...], preferred_element_type=jnp.float32) * dis
    )


def _final_body(p_ref, y_ref, degp_ref, b_ref, g_ref, be_ref, bat_ref,
                pw_ref, pb_ref, o_ref, sums, cnts):
    i = pl.program_id(0)

    @pl.when(i == 0)
    def _():
        sums[...] = jnp.zeros_like(sums)
        cnts[...] = jnp.zeros_like(cnts)

    dis = _dis_of(degp_ref[...])
    accv = p_ref[0] + p_ref[1] + y_ref[...]
    t = jnp.maximum(accv * dis + b_ref[...], 0.0)
    mu = jnp.mean(t, axis=-1, keepdims=True)
    var = jnp.mean((t - mu) ** 2, axis=-1, keepdims=True)
    h = (t - mu) / jnp.sqrt(var + 1e-5) * g_ref[...] + be_ref[...]

    bb = bat_ref[0]
    gid = lax.broadcasted_iota(jnp.int32, (NUM_GRAPHS, bb.shape[1]), 0)
    oh = (gid == bb).astype(jnp.float32)
    sums[...] += jnp.dot(oh, h, preferred_element_type=jnp.float32)
    cnts[...] += jnp.broadcast_to(
        jnp.sum(oh, axis=1, keepdims=True), (NUM_GRAPHS, D)
    )

    @pl.when(i == pl.num_programs(0) - 1)
    def _():
        pooled = sums[...] / jnp.maximum(cnts[...], 1.0)
        logits = (
            jnp.dot(pooled, pw_ref[...], preferred_element_type=jnp.float32)
            + pb_ref[...]
        )
        o_ref[...] = jax.nn.sigmoid(logits)


_R = 2528

_enc = pl.pallas_call(
    _enc_body,
    grid=(10,),
    in_specs=[
        pl.BlockSpec((1000, D), lambda i: (i, 0)),
        pl.BlockSpec((D, D), lambda i: (0, 0)),
        pl.BlockSpec((1, D), lambda i: (0, 0)),
    ],
    out_specs=pl.BlockSpec((1000, D), lambda i: (i, 0)),
    out_shape=jax.ShapeDtypeStruct((N_NODES, D), jnp.float32),
)

_pre = pl.pallas_call(
    _pre_body,
    grid=(NP // _R,),
    in_specs=[
        pl.BlockSpec((_R, D), lambda i: (i, 0)),
        pl.BlockSpec((D, D), lambda i: (0, 0)),
        pl.BlockSpec((NC, _R, 16), lambda i: (0, i, 0)),
    ],
    out_specs=pl.BlockSpec((_R, D), lambda i: (i, 0)),
    out_shape=jax.ShapeDtypeStruct((NP, D), jnp.float32),
)

_postpre = pl.pallas_call(
    _postpre_body,
    grid=(NP // _R,),
    in_specs=[
        pl.BlockSpec((NC, _R, D), lambda i: (0, i, 0)),
        pl.BlockSpec((_R, D), lambda i: (i, 0)),
        pl.BlockSpec((NC, _R, 16), lambda i: (0, i, 0)),
        pl.BlockSpec((1, D), lambda i: (0, 0)),
        pl.BlockSpec((1, D), lambda i: (0, 0)),
        pl.BlockSpec((1, D), lambda i: (0, 0)),
        pl.BlockSpec((D, D), lambda i: (0, 0)),
    ],
    out_specs=pl.BlockSpec((_R, D), lambda i: (i, 0)),
    out_shape=jax.ShapeDtypeStruct((NP, D), jnp.float32),
)

_RB = 1000

_final = pl.pallas_call(
    _final_body,
    grid=(N_NODES // _RB,),
    in_specs=[
        pl.BlockSpec((NC, _RB, D), lambda i: (0, i, 0)),
        pl.BlockSpec((_RB, D), lambda i: (i, 0)),
        pl.BlockSpec((NC, _RB, 16), lambda i: (0, i, 0)),
        pl.BlockSpec((1, D), lambda i: (0, 0)),
        pl.BlockSpec((1, D), lambda i: (0, 0)),
        pl.BlockSpec((1, D), lambda i: (0, 0)),
        pl.BlockSpec((1, 1, _RB), lambda i: (i, 0, 0)),
        pl.BlockSpec((D, D), lambda i: (0, 0)),
        pl.BlockSpec((1, D), lambda i: (0, 0)),
    ],
    out_specs=pl.BlockSpec((NUM_GRAPHS, D), lambda i: (0, 0)),
    out_shape=jax.ShapeDtypeStruct((NUM_GRAPHS, D), jnp.float32),
    scratch_shapes=[
        pltpu.VMEM((NUM_GRAPHS, D), jnp.float32),
        pltpu.VMEM((NUM_GRAPHS, D), jnp.float32),
    ],
)


def kernel(x, edge_index, batch, enc_W, enc_b, W1, b1, g1, be1,
           W2, b2, g2, be2, W3, b3, g3, be3, pred_W, pred_b):
    src = edge_index[0].astype(jnp.int32)
    dst = edge_index[1].astype(jnp.int32)
    pad = jnp.full((E_PAD - N_EDGES,), DUMMY, jnp.int32)
    src_p = jnp.concatenate([src, pad]).reshape(NTILES, CPT, CH)
    dst_p = jnp.concatenate([dst, pad]).reshape(NTILES, CPT, CH)
    zrows = jnp.zeros((ZR, D), jnp.float32)
    z16 = jnp.zeros((ZR, 16), jnp.float32)
    ones16 = jnp.ones((CH, 16), jnp.float32)

    _edge_scatter, _deg_scatter = _sc_kernels()
    degp = _deg_scatter(dst_p, z16, ones16)
    h = _enc(x, enc_W, enc_b.reshape(1, D))
    hp = jnp.pad(h, ((0, NP - N_NODES), (0, 0)))

    y = _pre(hp, W1, degp)
    P = _edge_scatter(y, src_p, dst_p, zrows)
    y = _postpre(P, y, degp, b1.reshape(1, D), g1.reshape(1, D),
                 be1.reshape(1, D), W2)
    P = _edge_scatter(y, src_p, dst_p, zrows)
    y = _postpre(P, y, degp, b2.reshape(1, D), g2.reshape(1, D),
                 be2.reshape(1, D), W3)
    P = _edge_scatter(y, src_p, dst_p, zrows)

    pWp = jnp.zeros((D, D), jnp.float32).at[:, :2].set(pred_W)
    pbp = jnp.zeros((1, D), jnp.float32).at[0, :2].set(pred_b)
    batp = batch.astype(jnp.int32).reshape(N_NODES // _RB, 1, _RB)
    probs = _final(P[:, :N_NODES], y[:N_NODES], degp[:, :N_NODES],
                   b3.reshape(1, D), g3.reshape(1, D), be3.reshape(1, D),
                   batp, pWp, pbp)
    return probs[:, :2]


# speedup vs baseline: 1.4199x; 1.0005x over previous
import functools

import jax
import jax.numpy as jnp
from jax import lax
from jax.experimental import pallas as pl
from jax.experimental.pallas import tpu as pltpu
from jax.experimental.pallas import tpu_sc as plsc

N_NODES = 10000
N_EDGES = 320000
D = 128
NUM_GRAPHS = 16

NC, NS = 2, 16
NTILES = NC * NS
CH = 128
CPT = 79
E_PAD = NTILES * CPT * CH
HS0 = (CPT + 1) // 2
HALVES = (HS0, CPT - HS0)
DUMMY = N_NODES
NP = 10112
ZR = NP // NS

@functools.cache
def _sc_kernels():
    mesh = plsc.VectorSubcoreMesh(
        core_axis_name="c", subcore_axis_name="s", num_cores=NC, num_subcores=NS
    )

    @functools.partial(
        pl.kernel,
        out_type=jax.ShapeDtypeStruct((NC, NP, D), jnp.float32),
        mesh=mesh,
        scratch_types=(
            pltpu.VMEM((CPT, CH), jnp.int32),
            pltpu.VMEM((CPT, CH), jnp.int32),
            pltpu.VMEM((CH, D), jnp.float32),
            pltpu.VMEM_SHARED((NP, D), jnp.float32),
            pltpu.SemaphoreType.DMA,
        ),
    )
    def _edge_scatter(y_hbm, src_hbm, dst_hbm, z_hbm, out_hbm,
                      srcs, dsts, rows, acc, sem):
        cid = lax.axis_index("c")
        sid = lax.axis_index("s")
        wid = cid * NS + sid
        r0 = sid * ZR
        pltpu.sync_copy(z_hbm, acc.at[pl.ds(r0, ZR)])
        pltpu.sync_copy(src_hbm.at[wid], srcs)
        pltpu.sync_copy(dst_hbm.at[wid], dsts)
        plsc.subcore_barrier()

        def body(j, carry):
            pltpu.async_copy(y_hbm.at[srcs.at[j]], rows, sem).wait()
            pltpu.sync_copy(rows, acc.at[dsts.at[j]], add=True)
            return carry

        lax.fori_loop(0, CPT, body, 0)
        plsc.subcore_barrier()
        pltpu.sync_copy(acc.at[pl.ds(r0, ZR)], out_hbm.at[cid, pl.ds(r0, ZR)])

    @functools.partial(
        pl.kernel,
        out_type=jax.ShapeDtypeStruct((NC, NP, 16), jnp.float32),
        mesh=mesh,
        scratch_types=(
            pltpu.VMEM((CPT, CH), jnp.int32),
            pltpu.VMEM((CH, 16), jnp.float32),
            pltpu.VMEM_SHARED((NP, 16), jnp.float32),
        ),
    )
    def _deg_scatter(dst_hbm, z_hbm, ones_hbm, out_hbm, dsts, ones, acc):
        cid = lax.axis_index("c")
        sid = lax.axis_index("s")
        wid = cid * NS + sid
        r0 = sid * ZR
        pltpu.sync_copy(z_hbm, acc.at[pl.ds(r0, ZR)])
        pltpu.sync_copy(dst_hbm.at[wid], dsts)
        pltpu.sync_copy(ones_hbm, ones)
        plsc.subcore_barrier()

        def body(j, carry):
            pltpu.sync_copy(ones, acc.at[dsts.at[j]], add=True)
            return carry

        lax.fori_loop(0, CPT, body, 0)
        plsc.subcore_barrier()
        pltpu.sync_copy(acc.at[pl.ds(r0, ZR)], out_hbm.at[cid, pl.ds(r0, ZR)])

    return _edge_scatter, _deg_scatter


def _dis_of(degp):
    deg = degp[0, :, 0:1] + degp[1, :, 0:1] + 1.0
    return 1.0 / jnp.sqrt(deg)


def _enc_body(x_ref, w_ref, b_ref, o_ref):
    v = jnp.dot(x_ref[...], w_ref[...], preferred_element_type=jnp.float32)
    v = v + b_ref[...]
    o_ref[...] = jnp.where(v > 0, v, jnp.exp(jnp.minimum(v, 0.0)) - 1.0)


def _pre_body(h_ref, w_ref, degp_ref, o_ref):
    dis = _dis_of(degp_ref[...])
    o_ref[...] = (
        jnp.dot(h_ref[...], w_ref[...], preferred_element_type=jnp.float32) * dis
    )


def _postpre_body(p_ref, y_ref, degp_ref, b_ref, g_ref, be_ref, w_ref, o_ref):
    dis = _dis_of(degp_ref[...])
    accv = p_ref[0] + p_ref[1] + y_ref[...]
    t = jnp.maximum(accv * dis + b_ref[...], 0.0)
    mu = jnp.mean(t, axis=-1, keepdims=True)
    var = jnp.mean((t - mu) ** 2, axis=-1, keepdims=True)
    h = (t - mu) / jnp.sqrt(var + 1e-5) * g_ref[...] + be_ref[...]
    o_ref[...] = (
        jnp.dot(h, w_ref[...], preferred_element_type=jnp.float32) * dis
    )


def _final_body(p_ref, y_ref, degp_ref, b_ref, g_ref, be_ref, bat_ref,
                pw_ref, pb_ref, o_ref, sums, cnts):
    i = pl.program_id(0)

    @pl.when(i == 0)
    def _():
        sums[...] = jnp.zeros_like(sums)
        cnts[...] = jnp.zeros_like(cnts)

    dis = _dis_of(degp_ref[...])
    accv = p_ref[0] + p_ref[1] + y_ref[...]
    t = jnp.maximum(accv * dis + b_ref[...], 0.0)
    mu = jnp.mean(t, axis=-1, keepdims=True)
    var = jnp.mean((t - mu) ** 2, axis=-1, keepdims=True)
    h = (t - mu) / jnp.sqrt(var + 1e-5) * g_ref[...] + be_ref[...]

    bb = bat_ref[0]
    gid = lax.broadcasted_iota(jnp.int32, (NUM_GRAPHS, bb.shape[1]), 0)
    oh = (gid == bb).astype(jnp.float32)
    sums[...] += jnp.dot(oh, h, preferred_element_type=jnp.float32)
    cnts[...] += jnp.broadcast_to(
        jnp.sum(oh, axis=1, keepdims=True), (NUM_GRAPHS, D)
    )

    @pl.when(i == pl.num_programs(0) - 1)
    def _():
        pooled = sums[...] / jnp.maximum(cnts[...], 1.0)
        logits = (
            jnp.dot(pooled, pw_ref[...], preferred_element_type=jnp.float32)
            + pb_ref[...]
        )
        o_ref[...] = jax.nn.sigmoid(logits)


_R = 2528

_enc = pl.pallas_call(
    _enc_body,
    grid=(10,),
    in_specs=[
        pl.BlockSpec((1000, D), lambda i: (i, 0)),
        pl.BlockSpec((D, D), lambda i: (0, 0)),
        pl.BlockSpec((1, D), lambda i: (0, 0)),
    ],
    out_specs=pl.BlockSpec((1000, D), lambda i: (i, 0)),
    out_shape=jax.ShapeDtypeStruct((N_NODES, D), jnp.float32),
)

_pre = pl.pallas_call(
    _pre_body,
    grid=(NP // _R,),
    in_specs=[
        pl.BlockSpec((_R, D), lambda i: (i, 0)),
        pl.BlockSpec((D, D), lambda i: (0, 0)),
        pl.BlockSpec((NC, _R, 16), lambda i: (0, i, 0)),
    ],
    out_specs=pl.BlockSpec((_R, D), lambda i: (i, 0)),
    out_shape=jax.ShapeDtypeStruct((NP, D), jnp.float32),
)

_postpre = pl.pallas_call(
    _postpre_body,
    grid=(NP // _R,),
    in_specs=[
        pl.BlockSpec((NC, _R, D), lambda i: (0, i, 0)),
        pl.BlockSpec((_R, D), lambda i: (i, 0)),
        pl.BlockSpec((NC, _R, 16), lambda i: (0, i, 0)),
        pl.BlockSpec((1, D), lambda i: (0, 0)),
        pl.BlockSpec((1, D), lambda i: (0, 0)),
        pl.BlockSpec((1, D), lambda i: (0, 0)),
        pl.BlockSpec((D, D), lambda i: (0, 0)),
    ],
    out_specs=pl.BlockSpec((_R, D), lambda i: (i, 0)),
    out_shape=jax.ShapeDtypeStruct((NP, D), jnp.float32),
)

_RB = 1000

_final = pl.pallas_call(
    _final_body,
    grid=(N_NODES // _RB,),
    in_specs=[
        pl.BlockSpec((NC, _RB, D), lambda i: (0, i, 0)),
        pl.BlockSpec((_RB, D), lambda i: (i, 0)),
        pl.BlockSpec((NC, _RB, 16), lambda i: (0, i, 0)),
        pl.BlockSpec((1, D), lambda i: (0, 0)),
        pl.BlockSpec((1, D), lambda i: (0, 0)),
        pl.BlockSpec((1, D), lambda i: (0, 0)),
        pl.BlockSpec((1, 1, _RB), lambda i: (i, 0, 0)),
        pl.BlockSpec((D, D), lambda i: (0, 0)),
        pl.BlockSpec((1, D), lambda i: (0, 0)),
    ],
    out_specs=pl.BlockSpec((NUM_GRAPHS, D), lambda i: (0, 0)),
    out_shape=jax.ShapeDtypeStruct((NUM_GRAPHS, D), jnp.float32),
    scratch_shapes=[
        pltpu.VMEM((NUM_GRAPHS, D), jnp.float32),
        pltpu.VMEM((NUM_GRAPHS, D), jnp.float32),
    ],
)


def kernel(x, edge_index, batch, enc_W, enc_b, W1, b1, g1, be1,
           W2, b2, g2, be2, W3, b3, g3, be3, pred_W, pred_b):
    src = edge_index[0].astype(jnp.int32)
    dst = edge_index[1].astype(jnp.int32)
    pad = jnp.full((E_PAD - N_EDGES,), DUMMY, jnp.int32)
    src_p = jnp.concatenate([src, pad]).reshape(NTILES, CPT, CH)
    dst_p = jnp.concatenate([dst, pad]).reshape(NTILES, CPT, CH)
    zrows = jnp.zeros((ZR, D), jnp.float32)
    z16 = jnp.zeros((ZR, 16), jnp.float32)
    ones16 = jnp.ones((CH, 16), jnp.float32)

    _edge_scatter, _deg_scatter = _sc_kernels()
    degp = _deg_scatter(dst_p, z16, ones16)
    h = _enc(x, enc_W, enc_b.reshape(1, D))
    hp = jnp.pad(h, ((0, NP - N_NODES), (0, 0)))

    y = _pre(hp, W1, degp)
    P = _edge_scatter(y, src_p, dst_p, zrows)
    y = _postpre(P, y, degp, b1.reshape(1, D), g1.reshape(1, D),
                 be1.reshape(1, D), W2)
    P = _edge_scatter(y, src_p, dst_p, zrows)
    y = _postpre(P, y, degp, b2.reshape(1, D), g2.reshape(1, D),
                 be2.reshape(1, D), W3)
    P = _edge_scatter(y, src_p, dst_p, zrows)

    pWp = jnp.zeros((D, D), jnp.float32).at[:, :2].set(pred_W)
    pbp = jnp.zeros((1, D), jnp.float32).at[0, :2].set(pred_b)
    batp = batch.astype(jnp.int32).reshape(N_NODES // _RB, 1, _RB)
    probs = _final(P[:, :N_NODES], y[:N_NODES], degp[:, :N_NODES],
                   b3.reshape(1, D), g3.reshape(1, D), be3.reshape(1, D),
                   batp, pWp, pbp)
    return probs[:, :2]
